# Initial kernel scaffold; baseline (speedup 1.0000x reference)
import jax
import jax.numpy as jnp
from jax.experimental import pallas as pl


def kernel(cls_pred, reg_pred, dir_pred, gt_boxes, batch_size):
    raise NotImplementedError("write your pallas kernel here")


# speedup vs baseline: 4.5179x; 4.5179x over previous
import jax
import jax.numpy as jnp
from jax.experimental import pallas as pl
from jax.experimental.pallas import tpu as pltpu

X_MIN, X_MAX = 0.0, 200.0
Y_MIN, Y_MAX = -50.0, 50.0
S = 0.4
INV_S = 2.5
ALPHA = 0.25
W_CLS, W_REG, W_DIR = 1.0, 2.0, 0.2
CW = (1.0, 5.0, 5.0)


def _ls_pair(x):
    sp = jnp.log1p(jnp.exp(-jnp.abs(x)))
    ls_pos = jnp.where(x >= 0, -sp, x - sp)
    return ls_pos, ls_pos - x


def _sigmoid_from(x):
    e = jnp.exp(-jnp.abs(x))
    r = 1.0 / (1.0 + e)
    return jnp.where(x >= 0, r, 1.0 - r)


def _loss_kernel(cls_ref, reg_ref, dir_ref, gt_ref, gtt_ref, out_ref):
    f32 = jnp.float32
    H, W = cls_ref.shape[2], cls_ref.shape[3]

    bt = gtt_ref[0]
    N = bt.shape[1]
    x, y, z = bt[0:1, :], bt[1:2, :], bt[2:3, :]
    l, w, h = bt[3:4, :], bt[4:5, :], bt[5:6, :]
    rot = bt[6:7, :]
    cid = bt[7:8, :].astype(jnp.int32)

    gxf = jnp.floor((x - X_MIN) * INV_S)
    gyf = jnp.floor((y - Y_MIN) * INV_S)
    gx = gxf.astype(jnp.int32)
    gy = gyf.astype(jnp.int32)
    valid = ((x >= X_MIN) & (x < X_MAX) & (y >= Y_MIN) & (y < Y_MAX)
             & (gx >= 0) & (gx < W) & (gy >= 0) & (gy < H))

    boxes = gt_ref[0]
    xc, yc = boxes[:, 0:1], boxes[:, 1:2]
    gxc = jnp.floor((xc - X_MIN) * INV_S).astype(jnp.int32)
    gyc = jnp.floor((yc - Y_MIN) * INV_S).astype(jnp.int32)
    valid_c = ((xc >= X_MIN) & (xc < X_MAX) & (yc >= Y_MIN) & (yc < Y_MAX)
               & (gxc >= 0) & (gxc < W) & (gyc >= 0) & (gyc < H))
    cell_row = gy * W + gx
    cell_col = gyc * W + gxc
    later = jax.lax.broadcasted_iota(jnp.int32, (N, N), 0) > \
        jax.lax.broadcasted_iota(jnp.int32, (N, N), 1)
    killed = jnp.any((cell_col == cell_row) & later & valid_c,
                     axis=0, keepdims=True)
    is_last = valid & jnp.logical_not(killed)

    row_io = jax.lax.broadcasted_iota(jnp.int32, (H, N), 0)
    col_io = jax.lax.broadcasted_iota(jnp.int32, (W, N), 0)
    rows_eq = row_io == gy
    cols_eq = col_io == gx
    rows_near = jnp.abs(row_io - gy) <= 1
    cols_near = jnp.abs(col_io - gx) <= 1

    cols_hit = jnp.where(cols_eq & valid, 1.0, 0.0).astype(f32)
    cols_nr = jnp.where(cols_near & valid, 1.0, 0.0).astype(f32)

    dn = (((1,), (1,)), ((), ()))

    def dot_nt(a, b):
        return jax.lax.dot_general(a, b, dn, preferred_element_type=f32)

    cls_num = jnp.zeros((), f32)
    vm_cnt = jnp.zeros((), f32)
    poscnt = None
    for c in range(3):
        mc = cid == c
        rows_hit_c = jnp.where(rows_eq & mc, 1.0, 0.0).astype(f32)
        rows_nr_c = jnp.where(rows_near & mc, 1.0, 0.0).astype(f32)
        hitcnt = dot_nt(rows_hit_c, cols_hit)
        nearcnt = dot_nt(rows_nr_c, cols_nr)
        hit = hitcnt > 0.0
        vmf = jnp.where(hit | (nearcnt == 0.0), 1.0, 0.0)

        xl = cls_ref[0, c]
        ls_pos, ls_neg = _ls_pair(xl)
        p = _sigmoid_from(xl)
        bce = jnp.where(hit, -ls_pos, -ls_neg)
        one_m = jnp.where(hit, 1.0 - p, p)
        fw = jnp.where(hit, ALPHA * CW[c], 1.0 - ALPHA) * one_m * one_m * one_m
        cls_num = cls_num + jnp.sum(fw * bce * vmf)
        vm_cnt = vm_cnt + jnp.sum(vmf)
        poscnt = hitcnt if poscnt is None else poscnt + hitcnt

    pos_f = jnp.where(poscnt > 0.0, 1.0, 0.0)
    pos_cnt = jnp.sum(pos_f)

    ccx = X_MIN + (gxf + 0.5) * S
    ccy = Y_MIN + (gyf + 0.5) * S
    rv = ((x - ccx) * INV_S, (y - ccy) * INV_S, z,
          jnp.log(jnp.maximum(l, 1e-3)), jnp.log(jnp.maximum(w, 1e-3)),
          jnp.log(jnp.maximum(h, 1e-3)), jnp.sin(rot))
    rows_last = jnp.where(rows_eq & is_last, 1.0, 0.0).astype(f32)
    reg_num = jnp.zeros((), f32)
    for i in range(7):
        colv = jnp.where(cols_eq, rv[i], 0.0).astype(f32)
        reg_t = dot_nt(rows_last, colv)
        d = reg_ref[0, i] - reg_t
        ad = jnp.abs(d)
        sl1 = jnp.where(ad < 1.0, 0.5 * d * d, ad - 0.5)
        reg_num = reg_num + jnp.sum(sl1 * pos_f)

    dbin0 = jnp.cos(rot) >= 0.0
    dir_num = jnp.zeros((), f32)
    for k in range(2):
        mk = dbin0 if k == 0 else jnp.logical_not(dbin0)
        rows_d = jnp.where(rows_eq & mk, 1.0, 0.0).astype(f32)
        tcnt = dot_nt(rows_d, cols_hit)
        tk = tcnt > 0.0
        xd = dir_ref[0, k]
        ls_pos, ls_neg = _ls_pair(xd)
        dbce = jnp.where(tk, -ls_pos, -ls_neg)
        dir_num = dir_num + jnp.sum(dbce * pos_f)

    lane = jax.lax.broadcasted_iota(jnp.int32, (1, 128), 1)
    vec = (jnp.where(lane == 0, cls_num, 0.0)
           + jnp.where(lane == 1, vm_cnt, 0.0)
           + jnp.where(lane == 2, reg_num, 0.0)
           + jnp.where(lane == 3, pos_cnt, 0.0)
           + jnp.where(lane == 4, dir_num, 0.0)).astype(f32)
    out_ref[0] = vec


def kernel(cls_pred, reg_pred, dir_pred, gt_boxes, batch_size):
    B, C, H, W = cls_pred.shape
    N = gt_boxes.shape[1]
    gtt = jnp.transpose(gt_boxes, (0, 2, 1))
    parts = pl.pallas_call(
        _loss_kernel,
        grid=(B,),
        in_specs=[
            pl.BlockSpec((1, C, H, W), lambda b: (b, 0, 0, 0)),
            pl.BlockSpec((1, 7, H, W), lambda b: (b, 0, 0, 0)),
            pl.BlockSpec((1, 2, H, W), lambda b: (b, 0, 0, 0)),
            pl.BlockSpec((1, N, 8), lambda b: (b, 0, 0)),
            pl.BlockSpec((1, 8, N), lambda b: (b, 0, 0)),
        ],
        out_specs=pl.BlockSpec((1, 1, 128), lambda b: (b, 0, 0)),
        out_shape=jax.ShapeDtypeStruct((B, 1, 128), jnp.float32),
        compiler_params=pltpu.CompilerParams(
            dimension_semantics=("parallel",),
        ),
    )(cls_pred, reg_pred, dir_pred, gt_boxes, gtt)
    s = jnp.sum(parts[:, 0, :], axis=0)
    cls_loss = s[0] / (s[1] + 1e-6)
    rcnt = s[3] * 7.0
    reg_loss = jnp.where(rcnt > 0, s[2] / jnp.maximum(rcnt, 1.0), 0.0)
    dcnt = s[3] * 2.0
    dir_loss = jnp.where(dcnt > 0, s[4] / jnp.maximum(dcnt, 1.0), 0.0)
    total = W_CLS * cls_loss + W_REG * reg_loss + W_DIR * dir_loss
    return jnp.stack([total, cls_loss, reg_loss, dir_loss])


# speedup vs baseline: 4.8144x; 1.0656x over previous
import jax
import jax.numpy as jnp
from jax.experimental import pallas as pl
from jax.experimental.pallas import tpu as pltpu

X_MIN, X_MAX = 0.0, 200.0
Y_MIN, Y_MAX = -50.0, 50.0
S = 0.4
INV_S = 2.5
ALPHA = 0.25
W_CLS, W_REG, W_DIR = 1.0, 2.0, 0.2
CW = (1.0, 5.0, 5.0)


def _ls_pair(x):
    sp = jnp.log1p(jnp.exp(-jnp.abs(x)))
    ls_pos = jnp.where(x >= 0, -sp, x - sp)
    return ls_pos, ls_pos - x


def _loss_kernel(cls_ref, reg_ref, dir_ref, gt_ref, out_ref, acc_ref):
    f32 = jnp.float32
    H, W = cls_ref.shape[2], cls_ref.shape[3]

    boxes = gt_ref[0]
    N = boxes.shape[0]
    bt = jnp.transpose(boxes)
    x, y, z = bt[0:1, :], bt[1:2, :], bt[2:3, :]
    l, w, h = bt[3:4, :], bt[4:5, :], bt[5:6, :]
    rot = bt[6:7, :]
    cid = bt[7:8, :].astype(jnp.int32)

    gxf = jnp.floor((x - X_MIN) * INV_S)
    gyf = jnp.floor((y - Y_MIN) * INV_S)
    gx = gxf.astype(jnp.int32)
    gy = gyf.astype(jnp.int32)
    valid = ((x >= X_MIN) & (x < X_MAX) & (y >= Y_MIN) & (y < Y_MAX)
             & (gx >= 0) & (gx < W) & (gy >= 0) & (gy < H))

    xc, yc = boxes[:, 0:1], boxes[:, 1:2]
    gxc = jnp.floor((xc - X_MIN) * INV_S).astype(jnp.int32)
    gyc = jnp.floor((yc - Y_MIN) * INV_S).astype(jnp.int32)
    valid_c = ((xc >= X_MIN) & (xc < X_MAX) & (yc >= Y_MIN) & (yc < Y_MAX)
               & (gxc >= 0) & (gxc < W) & (gyc >= 0) & (gyc < H))
    cell_row = gy * W + gx
    cell_col = gyc * W + gxc
    later = jax.lax.broadcasted_iota(jnp.int32, (N, N), 0) > \
        jax.lax.broadcasted_iota(jnp.int32, (N, N), 1)
    killed = jnp.any((cell_col == cell_row) & later & valid_c,
                     axis=0, keepdims=True)
    is_last = valid & jnp.logical_not(killed)

    row_io = jax.lax.broadcasted_iota(jnp.int32, (H, N), 0)
    col_io = jax.lax.broadcasted_iota(jnp.int32, (W, N), 0)
    rows_eq = row_io == gy
    cols_eq = col_io == gx
    rows_near = jnp.abs(row_io - gy) <= 1
    cols_near = jnp.abs(col_io - gx) <= 1

    cols_hit = jnp.where(cols_eq & valid, 1.0, 0.0).astype(f32)
    cols_nr = jnp.where(cols_near & valid, 1.0, 0.0).astype(f32)

    dn = (((1,), (1,)), ((), ()))

    def dot_nt(a, b):
        return jax.lax.dot_general(a, b, dn, preferred_element_type=f32)

    cls_acc = jnp.zeros((), f32)
    vm_cnt = jnp.zeros((), f32)
    poscnt = None
    for c in range(3):
        mc = cid == c
        rows_hit_c = jnp.where(rows_eq & mc, 1.0, 0.0).astype(f32)
        rows_nr_c = jnp.where(rows_near & mc, 1.0, 0.0).astype(f32)
        hitcnt = dot_nt(rows_hit_c, cols_hit)
        nearcnt = dot_nt(rows_nr_c, cols_nr)
        hit = hitcnt > 0.0
        vm = hit | (nearcnt == 0.0)

        xl = cls_ref[0, c]
        ls_pos, ls_neg = _ls_pair(xl)
        ls_a = jnp.where(hit, ls_pos, ls_neg)
        ls_b = jnp.where(hit, ls_neg, ls_pos)
        fac = jnp.where(hit, ALPHA * CW[c], 1.0 - ALPHA)
        term = fac * jnp.exp(3.0 * ls_b) * ls_a
        cls_acc = cls_acc + jnp.sum(jnp.where(vm, term, 0.0))
        vm_cnt = vm_cnt + jnp.sum(jnp.where(vm, 1.0, 0.0))
        poscnt = hitcnt if poscnt is None else poscnt + hitcnt

    pos = poscnt > 0.0
    pos_cnt = jnp.sum(jnp.where(pos, 1.0, 0.0))

    ccx = X_MIN + (gxf + 0.5) * S
    ccy = Y_MIN + (gyf + 0.5) * S
    rv = ((x - ccx) * INV_S, (y - ccy) * INV_S, z,
          jnp.log(jnp.maximum(l, 1e-3)), jnp.log(jnp.maximum(w, 1e-3)),
          jnp.log(jnp.maximum(h, 1e-3)), jnp.sin(rot))
    rows_last = jnp.where(rows_eq & is_last, 1.0, 0.0).astype(f32)
    reg_num = jnp.zeros((), f32)
    for i in range(7):
        colv = jnp.where(cols_eq, rv[i], 0.0).astype(f32)
        reg_t = dot_nt(rows_last, colv)
        d = reg_ref[0, i] - reg_t
        ad = jnp.abs(d)
        sl1 = jnp.where(ad < 1.0, 0.5 * d * d, ad - 0.5)
        reg_num = reg_num + jnp.sum(jnp.where(pos, sl1, 0.0))

    dbin0 = jnp.cos(rot) >= 0.0
    dir_acc = jnp.zeros((), f32)
    for k in range(2):
        mk = dbin0 if k == 0 else jnp.logical_not(dbin0)
        rows_d = jnp.where(rows_eq & mk, 1.0, 0.0).astype(f32)
        tk = dot_nt(rows_d, cols_hit) > 0.0
        ls_pos, ls_neg = _ls_pair(dir_ref[0, k])
        dsel = jnp.where(tk, ls_pos, ls_neg)
        dir_acc = dir_acc + jnp.sum(jnp.where(pos, dsel, 0.0))

    lane = jax.lax.broadcasted_iota(jnp.int32, (1, 128), 1)
    vec = (jnp.where(lane == 0, cls_acc, 0.0)
           + jnp.where(lane == 1, vm_cnt, 0.0)
           + jnp.where(lane == 2, reg_num, 0.0)
           + jnp.where(lane == 3, pos_cnt, 0.0)
           + jnp.where(lane == 4, dir_acc, 0.0)).astype(f32)

    b = pl.program_id(0)
    nb = pl.num_programs(0)

    @pl.when(b == 0)
    def _():
        acc_ref[...] = vec

    @pl.when(b > 0)
    def _():
        acc_ref[...] = acc_ref[...] + vec

    @pl.when(b == nb - 1)
    def _():
        s = acc_ref[...]
        cls_loss = -s[0, 0] / (s[0, 1] + 1e-6)
        rcnt = s[0, 3] * 7.0
        reg_loss = jnp.where(rcnt > 0, s[0, 2] / jnp.maximum(rcnt, 1.0), 0.0)
        dcnt = s[0, 3] * 2.0
        dir_loss = jnp.where(dcnt > 0, -s[0, 4] / jnp.maximum(dcnt, 1.0), 0.0)
        total = W_CLS * cls_loss + W_REG * reg_loss + W_DIR * dir_loss
        l4 = jax.lax.broadcasted_iota(jnp.int32, (1, 1, 4), 2)
        out_ref[...] = jnp.where(
            l4 == 0, total,
            jnp.where(l4 == 1, cls_loss,
                      jnp.where(l4 == 2, reg_loss, dir_loss))).astype(f32)


def kernel(cls_pred, reg_pred, dir_pred, gt_boxes, batch_size):
    B, C, H, W = cls_pred.shape
    N = gt_boxes.shape[1]
    out = pl.pallas_call(
        _loss_kernel,
        grid=(B,),
        in_specs=[
            pl.BlockSpec((1, C, H, W), lambda b: (b, 0, 0, 0)),
            pl.BlockSpec((1, 7, H, W), lambda b: (b, 0, 0, 0)),
            pl.BlockSpec((1, 2, H, W), lambda b: (b, 0, 0, 0)),
            pl.BlockSpec((1, N, 8), lambda b: (b, 0, 0)),
        ],
        out_specs=pl.BlockSpec((1, 1, 4), lambda b: (0, 0, 0)),
        out_shape=jax.ShapeDtypeStruct((1, 1, 4), jnp.float32),
        scratch_shapes=[pltpu.VMEM((1, 128), jnp.float32)],
        compiler_params=pltpu.CompilerParams(
            dimension_semantics=("arbitrary",),
        ),
    )(cls_pred, reg_pred, dir_pred, gt_boxes)
    return out.reshape(4)
